# Initial kernel scaffold; baseline (speedup 1.0000x reference)
import jax
import jax.numpy as jnp
from jax.experimental import pallas as pl


def kernel(c_seq):
    raise NotImplementedError("write your pallas kernel here")


# speedup vs baseline: 36.3652x; 36.3652x over previous
import jax
import jax.numpy as jnp
from jax import lax
from jax.experimental import pallas as pl
from jax.experimental.pallas import tpu as pltpu
from jax.experimental.pallas import tpu_sc as plsc

N = 4096
T = 2048
ROW_W = 2 * T
NC = 2
NS = 16
NW = NC * NS
ROWS_PER_W = N // NW
PRE_PAIRS = 256
PRE_W = 2 * PRE_PAIRS
N_CHUNKS = PRE_PAIRS // 16
REM_W = ROW_W - PRE_W
REM_CHUNKS = (T - PRE_PAIRS) // 16
TAN_PI_8 = 0.4142135623730951

def _bins16(x, y):
    one_i = jnp.full((16,), 1, jnp.int32)
    zero_i = jnp.full((16,), 0, jnp.int32)
    ax = jnp.abs(x)
    ay = jnp.abs(y)
    c1 = ay > ax * TAN_PI_8
    c2 = ay >= ax
    c3 = ay * TAN_PI_8 > ax
    s = (jnp.where(c1, one_i, zero_i) + jnp.where(c2, one_i, zero_i)
         + jnp.where(c3, one_i, zero_i))
    u = jnp.where(x >= 0.0, 8 + s, 15 - s)
    return jnp.where(y >= 0.0, u, 15 - u)


def _sc_body(x_hbm, out_hbm, buf, rembuf, hist, acc):
    c = lax.axis_index("c")
    s = lax.axis_index("s")
    wid = s * NC + c
    base = wid * ROWS_PER_W

    pltpu.sync_copy(x_hbm.at[pl.ds(base, ROWS_PER_W), pl.ds(0, PRE_W)], buf)

    acc[...] = jnp.zeros((16,), jnp.int32)
    iota = lax.iota(jnp.int32, 16)
    iota2 = 2 * iota
    perm = jnp.bitwise_and(iota2, 15)
    hi_lane = iota >= 8

    def _deinterleave(v0, v1):
        xv = jnp.where(hi_lane,
                       v1.at[perm].get(mode="promise_in_bounds"),
                       v0.at[perm].get(mode="promise_in_bounds"))
        yv = jnp.where(hi_lane,
                       v1.at[perm + 1].get(mode="promise_in_bounds"),
                       v0.at[perm + 1].get(mode="promise_in_bounds"))
        return xv, yv

    one_i = jnp.full((16,), 1, jnp.int32)
    rot_perms = [jnp.bitwise_and(iota + k, 15) for k in (8, 4, 2, 1)]

    def _or_all_lanes(m):
        for p in rot_perms:
            m = m | m.at[p].get(mode="promise_in_bounds")
        return m

    def row_body(r, carry):
        m = jnp.full((16,), 0, jnp.int32)
        for ch in range(N_CHUNKS):
            v0 = buf[r, pl.ds(ch * 32, 16)]
            v1 = buf[r, pl.ds(ch * 32 + 16, 16)]
            xv, yv = _deinterleave(v0, v1)
            m = m | (one_i << _bins16(xv, yv))
        m_all = _or_all_lanes(m)
        hist[...] = m_all
        mask = m_all[0]

        @pl.when(mask != 0xFFFF)
        def _fallback():
            pltpu.sync_copy(x_hbm.at[base + r, pl.ds(PRE_W, REM_W)], rembuf)

            def rem_body(ch, mm):
                v0 = rembuf[pl.ds(ch * 32, 16)]
                v1 = rembuf[pl.ds(ch * 32 + 16, 16)]
                xv, yv = _deinterleave(v0, v1)
                return mm | (one_i << _bins16(xv, yv))

            hist[...] = lax.fori_loop(0, REM_CHUNKS, rem_body, hist[...])

        ind = (_or_all_lanes(hist[...]) >> iota) & 1
        acc[...] = acc[...] + ind
        return carry

    lax.fori_loop(0, ROWS_PER_W, row_body, 0)
    pltpu.sync_copy(acc, out_hbm.at[wid])


def kernel(c_seq):
    xflat = c_seq.reshape(N, ROW_W)
    mesh = plsc.VectorSubcoreMesh(core_axis_name="c", subcore_axis_name="s")
    partial_occ = pl.kernel(
        _sc_body,
        out_type=jax.ShapeDtypeStruct((NW, 16), jnp.int32),
        mesh=mesh,
        scratch_types=[
            pltpu.VMEM((ROWS_PER_W, PRE_W), jnp.float32),
            pltpu.VMEM((REM_W,), jnp.float32),
            pltpu.VMEM((16,), jnp.int32),
            pltpu.VMEM((16,), jnp.int32),
        ],
        compiler_params=pltpu.CompilerParams(use_tc_tiling_on_sc=False),
    )(xflat)
    total = jnp.sum(partial_occ).astype(jnp.float32)
    return jnp.float32(1.0) - total / jnp.float32(N * 16)


# speedup vs baseline: 251.7343x; 6.9224x over previous
import jax
import jax.numpy as jnp
from jax import lax
from jax.experimental import pallas as pl
from jax.experimental.pallas import tpu as pltpu
from jax.experimental.pallas import tpu_sc as plsc

N = 4096
T = 2048
LANES = 128
NTILES = T // LANES
NC = 2
NS = 16
NW = NC * NS
ROWS_PER_W = N // NW
REM_TILES = NTILES - 1
N_BATCH = 4
BATCH_ROWS = ROWS_PER_W // N_BATCH
BLK = 16
TAN_PI_8 = 0.4142135623730951
FULL = 0xFFFF


def _bin_bits(x, y):
    ax = jnp.abs(x)
    ay = jnp.abs(y)
    c1 = ay > ax * TAN_PI_8
    c2 = ay >= ax
    c3 = ay * TAN_PI_8 > ax
    p = jnp.where(c3, 8, jnp.where(c2, 4, jnp.where(c1, 2, 1)))
    p = jnp.where(x < 0.0, p << 4, p)
    return jnp.where(y < 0.0, p << 8, p)


def _sc_body(x_hbm, out_hbm, buf, rembuf, blockmask, acc, *sems):
    c = lax.axis_index("c")
    s = lax.axis_index("s")
    wid = s * NC + c
    base = wid * ROWS_PER_W

    copies = [
        pltpu.async_copy(
            x_hbm.at[pl.ds(base + b * BATCH_ROWS, BATCH_ROWS), 0, :, :],
            buf.at[pl.ds(b * BATCH_ROWS, BATCH_ROWS)],
            sems[b],
        )
        for b in range(N_BATCH)
    ]

    iota = lax.iota(jnp.int32, 16)
    rot_perms = [jnp.bitwise_and(iota + k, 15) for k in (8, 4, 2, 1)]

    def _or_all_lanes(m):
        for p in rot_perms:
            m = m | m.at[p].get(mode="promise_in_bounds")
        return m

    def _prefix_mask(r):
        m = jnp.full((16,), 0, jnp.int32)
        for k in range(LANES // 16):
            xv = buf[r, 0, pl.ds(k * 16, 16)]
            yv = buf[r, 1, pl.ds(k * 16, 16)]
            m = m | _bin_bits(xv, yv)
        return _or_all_lanes(m)

    acc[...] = jnp.zeros((16,), jnp.int32)

    def block_fn(start):
        def row_fn(r, carry):
            and_m, accv = carry
            m_all = _prefix_mask(r)
            blockmask[r - start] = m_all
            ind = (m_all >> iota) & 1
            return and_m & m_all, accv + ind

        and_m, accv = lax.fori_loop(
            start, start + BLK, row_fn,
            (jnp.full((16,), -1, jnp.int32), jnp.zeros((16,), jnp.int32)))
        acc[...] = acc[...] + accv

        @pl.when(and_m[0] != FULL)
        def _rescan():
            def fix_fn(i, carry2):
                mv = blockmask[i]

                @pl.when(mv[0] != FULL)
                def _finish_row():
                    pltpu.sync_copy(
                        x_hbm.at[base + start + i, pl.ds(1, REM_TILES), :, :],
                        rembuf)

                    def rem_body(jj, mm):
                        for k in range(LANES // 16):
                            xv = rembuf[jj, 0, pl.ds(k * 16, 16)]
                            yv = rembuf[jj, 1, pl.ds(k * 16, 16)]
                            mm = mm | _bin_bits(xv, yv)
                        return mm

                    m2 = _or_all_lanes(
                        lax.fori_loop(0, REM_TILES, rem_body, mv))
                    delta = ((m2 >> iota) & 1) - ((mv >> iota) & 1)
                    acc[...] = acc[...] + delta

                return carry2

            lax.fori_loop(0, BLK, fix_fn, 0)

    for b in range(N_BATCH):
        copies[b].wait()
        for blk in range(BATCH_ROWS // BLK):
            block_fn(b * BATCH_ROWS + blk * BLK)

    pltpu.sync_copy(acc, out_hbm.at[wid])


def kernel(c_seq):
    x4 = c_seq.reshape(N, NTILES, LANES, 2).transpose(0, 1, 3, 2)
    mesh = plsc.VectorSubcoreMesh(core_axis_name="c", subcore_axis_name="s")
    partial_occ = pl.kernel(
        _sc_body,
        out_type=jax.ShapeDtypeStruct((NW, 16), jnp.int32),
        mesh=mesh,
        scratch_types=[
            pltpu.VMEM((ROWS_PER_W, 2, LANES), jnp.float32),
            pltpu.VMEM((REM_TILES, 2, LANES), jnp.float32),
            pltpu.VMEM((BLK, 16), jnp.int32),
            pltpu.VMEM((16,), jnp.int32),
        ] + [pltpu.SemaphoreType.DMA] * N_BATCH,
        compiler_params=pltpu.CompilerParams(use_tc_tiling_on_sc=False),
    )(x4)
    total = jnp.sum(partial_occ).astype(jnp.float32)
    return jnp.float32(1.0) - total / jnp.float32(N * 16)


# speedup vs baseline: 282.1364x; 1.1208x over previous
import jax
import jax.numpy as jnp
from jax import lax
from jax.experimental import pallas as pl
from jax.experimental.pallas import tpu as pltpu
from jax.experimental.pallas import tpu_sc as plsc

N = 4096
T = 2048
LANES = 128
NTILES = T // LANES
NC = 2
NS = 16
NW = NC * NS
ROWS_PER_W = N // NW
REM_TILES = NTILES - 1
HALF_ROWS = ROWS_PER_W // 2
TAN_PI_8 = 0.4142135623730951
FULL = 0xFFFF


def _bin_bits(x, y):
    ax = jnp.abs(x)
    ay = jnp.abs(y)
    c1 = ay > ax * TAN_PI_8
    c2 = ay >= ax
    c3 = ay * TAN_PI_8 > ax
    p = jnp.where(c3, 8, jnp.where(c2, 4, jnp.where(c1, 2, 1)))
    p = jnp.where(x < 0.0, p << 4, p)
    return jnp.where(y < 0.0, p << 8, p)


def _sc_body(x_hbm, out_hbm, buf, rembuf, acc, sem0, sem1):
    c = lax.axis_index("c")
    s = lax.axis_index("s")
    wid = s * NC + c
    base = wid * ROWS_PER_W

    cp0 = pltpu.async_copy(
        x_hbm.at[pl.ds(base, HALF_ROWS), 0, :, :],
        buf.at[pl.ds(0, HALF_ROWS)], sem0)
    cp1 = pltpu.async_copy(
        x_hbm.at[pl.ds(base + HALF_ROWS, HALF_ROWS), 0, :, :],
        buf.at[pl.ds(HALF_ROWS, HALF_ROWS)], sem1)

    acc[...] = jnp.zeros((16,), jnp.int32)
    iota = lax.iota(jnp.int32, 16)
    rot_perms = [jnp.bitwise_and(iota + k, 15) for k in (8, 4, 2, 1)]

    def _or_all_lanes(m):
        for p in rot_perms:
            m = m | m.at[p].get(mode="promise_in_bounds")
        return m

    def row_body(r, carry):
        m = jnp.full((16,), 0, jnp.int32)
        for k in range(LANES // 16):
            xv = buf[r, 0, pl.ds(k * 16, 16)]
            yv = buf[r, 1, pl.ds(k * 16, 16)]
            m = m | _bin_bits(xv, yv)
        m_all = _or_all_lanes(m)
        acc[...] = acc[...] + ((m_all >> iota) & 1)

        @pl.when(m_all[0] != FULL)
        def _fallback():
            pltpu.sync_copy(
                x_hbm.at[base + r, pl.ds(1, REM_TILES), :, :], rembuf)

            def rem_body(jj, mm):
                for k in range(LANES // 16):
                    xv = rembuf[jj, 0, pl.ds(k * 16, 16)]
                    yv = rembuf[jj, 1, pl.ds(k * 16, 16)]
                    mm = mm | _bin_bits(xv, yv)
                return mm

            m2 = _or_all_lanes(lax.fori_loop(0, REM_TILES, rem_body, m_all))
            acc[...] = acc[...] + (((m2 >> iota) & 1) - ((m_all >> iota) & 1))

        return carry

    cp0.wait()
    lax.fori_loop(0, HALF_ROWS, row_body, 0)
    cp1.wait()
    lax.fori_loop(HALF_ROWS, ROWS_PER_W, row_body, 0)

    pltpu.sync_copy(acc, out_hbm.at[wid])


def kernel(c_seq):
    x4 = c_seq.reshape(N, NTILES, LANES, 2).transpose(0, 1, 3, 2)
    mesh = plsc.VectorSubcoreMesh(core_axis_name="c", subcore_axis_name="s")
    partial_occ = pl.kernel(
        _sc_body,
        out_type=jax.ShapeDtypeStruct((NW, 16), jnp.int32),
        mesh=mesh,
        scratch_types=[
            pltpu.VMEM((ROWS_PER_W, 2, LANES), jnp.float32),
            pltpu.VMEM((REM_TILES, 2, LANES), jnp.float32),
            pltpu.VMEM((16,), jnp.int32),
            pltpu.SemaphoreType.DMA,
            pltpu.SemaphoreType.DMA,
        ],
        compiler_params=pltpu.CompilerParams(use_tc_tiling_on_sc=False),
    )(x4)
    total = jnp.sum(partial_occ).astype(jnp.float32)
    return jnp.float32(1.0) - total / jnp.float32(N * 16)
